# baseline (speedup 1.0000x reference)
import functools

import jax
import jax.numpy as jnp
from jax import lax
from jax.experimental import pallas as pl
from jax.experimental.pallas import tpu as pltpu
from jax.experimental.pallas import tpu_sc as plsc

V = 10000
E = 160000
D = 128
NH = 8
KQ = 16
MSG = 16
DE = 16
ACC_W = NH * MSG + NH

NC = 2
NS = 16
NW = NC * NS
EPW = E // NW
C = 40
NCHUNK = EPW // C
SB = 16
NB = 8
ROWS_PT = 632
VP = NS * ROWS_PT

_EPS = 1e-3


def _node_proj_body(h_ref, w_ref, g_ref, b_ref, kv_ref, q_ref):
    h = h_ref[...]
    mean = jnp.mean(h, axis=1, keepdims=True)
    var = jnp.mean((h - mean) ** 2, axis=1, keepdims=True)
    hn = g_ref[...] * (h - mean) * lax.rsqrt(var + _EPS) + b_ref[...]
    kvq = jnp.dot(hn, w_ref[...], preferred_element_type=jnp.float32)
    kv_ref[...] = kvq[:, : 2 * D]
    q_ref[...] = kvq[:, 2 * D :]


def _ef_body(e_ref, w_ref, out_ref):
    out_ref[...] = jnp.dot(e_ref[...], w_ref[...],
                           preferred_element_type=jnp.float32)


def _post_body(acc_ref, h_ref, wcomb_ref, e8_ref, g_ref, b_ref, rz1_ref,
               rz2_ref, wff1_ref, wff2_ref, out_ref):
    acc = acc_ref[0] + acc_ref[1]
    uagg = acc[:, :D]
    den = acc[:, D:]
    den = jnp.where(den == 0.0, 1.0, den)
    bcast = jnp.dot(1.0 / den, e8_ref[...], preferred_element_type=jnp.float32)
    agg = uagg * bcast
    mp = jnp.dot(agg, wcomb_ref[...], preferred_element_type=jnp.float32)
    h1 = h_ref[...] + rz1_ref[...] * mp
    mean = jnp.mean(h1, axis=1, keepdims=True)
    var = jnp.mean((h1 - mean) ** 2, axis=1, keepdims=True)
    h1n = g_ref[...] * (h1 - mean) * lax.rsqrt(var + _EPS) + b_ref[...]
    x = jnp.dot(h1n, wff1_ref[...], preferred_element_type=jnp.float32)
    ff = 0.5 * x * (1.0 + lax.erf(x * (2.0 ** -0.5)))
    out_ref[...] = h1 + rz2_ref[...] * jnp.dot(
        ff, wff2_ref[...], preferred_element_type=jnp.float32)


def _sc_attn_body(kv_hbm, q_hbm, ef_hbm, src_hbm, dst_hbm, zero_hbm, out_hbm,
                  srcS, dstS, kv0, kv1, q0, q1, efB, msg0, acc_sh,
                  sg0, sg1, se):
    cid = lax.axis_index("c")
    sid = lax.axis_index("s")
    wid = cid * NS + sid
    r0 = sid * ROWS_PT
    pltpu.sync_copy(zero_hbm.at[pl.ds(r0, ROWS_PT)],
                    acc_sh.at[pl.ds(r0, ROWS_PT)])
    plsc.subcore_barrier()

    pltpu.sync_copy(src_hbm.at[wid].at[pl.ds(0, SB)], srcS.at[0])
    pltpu.sync_copy(dst_hbm.at[wid].at[pl.ds(0, SB)], dstS.at[0])
    pltpu.sync_copy(src_hbm.at[wid].at[pl.ds(SB, SB)], srcS.at[1])
    pltpu.sync_copy(dst_hbm.at[wid].at[pl.ds(SB, SB)], dstS.at[1])

    def src_row(j):
        return srcS.at[(j // SB) % 2].at[j % SB]

    def dst_row(j):
        return dstS.at[(j // SB) % 2].at[j % SB]

    gbufs = ((kv0, q0, sg0), (kv1, q1, sg1))
    ebase = wid * EPW

    def issue_g(j, b):
        kvb, qb, sem = gbufs[b]
        pltpu.async_copy(kv_hbm.at[src_row(j)], kvb, sem)
        pltpu.async_copy(q_hbm.at[dst_row(j)], qb, sem)

    def wait_g(j, b):
        kvb, qb, sem = gbufs[b]
        pltpu.make_async_copy(kv_hbm.at[src_row(j)], kvb, sem).wait()
        pltpu.make_async_copy(q_hbm.at[dst_row(j)], qb, sem).wait()

    def issue_ef(j):
        pltpu.async_copy(ef_hbm.at[pl.ds(ebase + j * C, C)], efB, se)

    def wait_ef(j):
        pltpu.make_async_copy(ef_hbm.at[pl.ds(ebase + j * C, C)], efB,
                              se).wait()

    iota = lax.iota(jnp.int32, 16)
    perms = [iota ^ 8, iota ^ 4, iota ^ 2, iota ^ 1]
    lanesel = [iota == 8 + h for h in range(NH)]
    bidx = [jnp.full((16,), 8 + h, jnp.int32) for h in range(NH)]

    def compute(b, msgB):
        kvb, qb, _ = gbufs[b]

        def edge_body(e, c_):
            sv = jnp.zeros((16,), jnp.float32)
            avs = []
            for h in range(NH):
                kvv = kvb[e, pl.ds(h * KQ, 16)]
                efv = efB[e, pl.ds(h * KQ, 16)]
                qv = qb[e, pl.ds(h * KQ, 16)]
                p = (kvv + efv) * qv
                for pm in perms:
                    p = p + jnp.take(p, pm)
                a = jnp.exp(p)
                avs.append(a)
                sv = jnp.where(lanesel[h], a, sv)
            msgB[e, pl.ds(120, 16)] = sv
            for h in range(NH):
                vv = kvb[e, pl.ds(D + h * MSG, 16)]
                msgB[e, pl.ds(h * MSG, 16)] = avs[h] * vv
            return c_

        lax.fori_loop(0, C, edge_body, 0, unroll=4)

    issue_g(0, 0)
    issue_g(1, 1)
    issue_ef(0)

    def pair_body(p, carry):
        for b in range(2):
            j = p * 2 + b

            @pl.when(j < NCHUNK)
            def _():
                wait_g(j, b)
                wait_ef(j)

                compute(b, msg0)
                pltpu.sync_copy(msg0, acc_sh.at[dst_row(j)], add=True)

                nxt = j // SB + 1

                @pl.when(jnp.logical_and(j % SB == 0,
                                         jnp.logical_and(j > 0, nxt < NB)))
                def _():
                    pltpu.sync_copy(src_hbm.at[wid].at[pl.ds(nxt * SB, SB)],
                                    srcS.at[nxt % 2])
                    pltpu.sync_copy(dst_hbm.at[wid].at[pl.ds(nxt * SB, SB)],
                                    dstS.at[nxt % 2])

                @pl.when(j + 1 < NCHUNK)
                def _():
                    issue_ef(j + 1)

                @pl.when(j + 2 < NCHUNK)
                def _():
                    issue_g(j + 2, b)

        return carry

    lax.fori_loop(0, (NCHUNK + 2) // 2, pair_body, 0)
    plsc.subcore_barrier()
    pltpu.sync_copy(acc_sh.at[pl.ds(r0, ROWS_PT)],
                    out_hbm.at[cid].at[pl.ds(r0, ROWS_PT)])


@functools.cache
def _build_sc_attn():
    return pl.kernel(
        _sc_attn_body,
        out_type=jax.ShapeDtypeStruct((NC, VP, ACC_W), jnp.float32),
        mesh=plsc.VectorSubcoreMesh(core_axis_name="c", subcore_axis_name="s",
                                    num_cores=NC, num_subcores=NS),
        compiler_params=pltpu.CompilerParams(use_tc_tiling_on_sc=False,
                                             needs_layout_passes=False),
        scratch_types=[
            pltpu.VMEM((2, SB, C), jnp.int32),
            pltpu.VMEM((2, SB, C), jnp.int32),
            pltpu.VMEM((C, 2 * D), jnp.float32),
            pltpu.VMEM((C, 2 * D), jnp.float32),
            pltpu.VMEM((C, D), jnp.float32),
            pltpu.VMEM((C, D), jnp.float32),
            pltpu.VMEM((C, D), jnp.float32),
            pltpu.VMEM((C, ACC_W), jnp.float32),
            pltpu.VMEM_SHARED((VP, ACC_W), jnp.float32),
            pltpu.SemaphoreType.DMA,
            pltpu.SemaphoreType.DMA,
            pltpu.SemaphoreType.DMA,
        ],
    )


def _sc_attn(kv, q, ef, src, dst, zeros):
    return _build_sc_attn()(kv, q, ef, src, dst, zeros)


def kernel(node_embeddings, adjacency_list, edge_features, W_kqv, bm, ba,
           W_ef, W_comb, g1, b1, g2, b2, rz1, rz2, W_ff1, W_ff2):
    del bm, ba
    f32 = jnp.float32
    w3 = W_kqv.reshape(D, NH, 2 * KQ + MSG)
    wk = w3[:, :, :KQ].reshape(D, NH * KQ)
    wq = w3[:, :, KQ : 2 * KQ].reshape(D, NH * KQ)
    wv = w3[:, :, 2 * KQ :].reshape(D, NH * MSG)
    wkvq = jnp.concatenate([wk, wv, wq], axis=1)

    B1 = 1000
    kv, q = pl.pallas_call(
        _node_proj_body,
        grid=(V // B1,),
        in_specs=[
            pl.BlockSpec((B1, D), lambda i: (i, 0)),
            pl.BlockSpec((D, 3 * D), lambda i: (0, 0)),
            pl.BlockSpec((1, D), lambda i: (0, 0)),
            pl.BlockSpec((1, D), lambda i: (0, 0)),
        ],
        out_specs=[
            pl.BlockSpec((B1, 2 * D), lambda i: (i, 0)),
            pl.BlockSpec((B1, D), lambda i: (i, 0)),
        ],
        out_shape=[
            jax.ShapeDtypeStruct((V, 2 * D), f32),
            jax.ShapeDtypeStruct((V, D), f32),
        ],
    )(node_embeddings, wkvq, g1.reshape(1, D), b1.reshape(1, D))

    BE = 2000
    ef = pl.pallas_call(
        _ef_body,
        grid=(E // BE,),
        in_specs=[
            pl.BlockSpec((BE, DE), lambda i: (i, 0)),
            pl.BlockSpec((DE, NH * KQ), lambda i: (0, 0)),
        ],
        out_specs=pl.BlockSpec((BE, NH * KQ), lambda i: (i, 0)),
        out_shape=jax.ShapeDtypeStruct((E, NH * KQ), f32),
    )(edge_features, W_ef)

    pad_rows = NB * SB - NCHUNK
    src = jnp.pad(adjacency_list[:, 0].astype(jnp.int32).reshape(
        NW, NCHUNK, C), ((0, 0), (0, pad_rows), (0, 0)))
    dst = jnp.pad(adjacency_list[:, 1].astype(jnp.int32).reshape(
        NW, NCHUNK, C), ((0, 0), (0, pad_rows), (0, 0)))
    zeros = jnp.zeros((VP, ACC_W), f32)
    acc = _sc_attn(kv, q, ef, src, dst, zeros)[:, :V]

    e8 = jnp.kron(jnp.eye(NH, dtype=f32), jnp.ones((1, MSG), f32))
    B3 = 1000
    out = pl.pallas_call(
        _post_body,
        grid=(V // B3,),
        in_specs=[
            pl.BlockSpec((NC, B3, ACC_W), lambda i: (0, i, 0)),
            pl.BlockSpec((B3, D), lambda i: (i, 0)),
            pl.BlockSpec((NH * MSG, D), lambda i: (0, 0)),
            pl.BlockSpec((NH, D), lambda i: (0, 0)),
            pl.BlockSpec((1, D), lambda i: (0, 0)),
            pl.BlockSpec((1, D), lambda i: (0, 0)),
            pl.BlockSpec((1, D), lambda i: (0, 0)),
            pl.BlockSpec((1, D), lambda i: (0, 0)),
            pl.BlockSpec((D, 2 * D), lambda i: (0, 0)),
            pl.BlockSpec((2 * D, D), lambda i: (0, 0)),
        ],
        out_specs=pl.BlockSpec((B3, D), lambda i: (i, 0)),
        out_shape=jax.ShapeDtypeStruct((V, D), f32),
    )(acc, node_embeddings, W_comb, e8, g2.reshape(1, D), b2.reshape(1, D),
      rz1.reshape(1, D), rz2.reshape(1, D), W_ff1, W_ff2)
    return out


# speedup vs baseline: 1.0477x; 1.0477x over previous
import functools

import jax
import jax.numpy as jnp
from jax import lax
from jax.experimental import pallas as pl
from jax.experimental.pallas import tpu as pltpu
from jax.experimental.pallas import tpu_sc as plsc

V = 10000
E = 160000
D = 128
NH = 8
KQ = 16
MSG = 16
DE = 16
ACC_W = NH * MSG + NH

NC = 2
NS = 16
NW = NC * NS
EPW = E // NW
C = 40
NCHUNK = EPW // C
SB = 16
NB = 8
ROWS_PT = 632
VP = NS * ROWS_PT

_EPS = 1e-3


def _proj_body(h_ref, w_ref, g_ref, b_ref, e_ref, wef_ref,
               kv_ref, q_ref, ef_ref):
    @pl.when(pl.program_id(0) < 10)
    def _():
        h = h_ref[...]
        mean = jnp.mean(h, axis=1, keepdims=True)
        var = jnp.mean((h - mean) ** 2, axis=1, keepdims=True)
        hn = g_ref[...] * (h - mean) * lax.rsqrt(var + _EPS) + b_ref[...]
        kvq = jnp.dot(hn, w_ref[...], preferred_element_type=jnp.float32)
        kv_ref[...] = kvq[:, : 2 * D]
        q_ref[...] = kvq[:, 2 * D :]

    ef_ref[...] = jnp.dot(e_ref[...], wef_ref[...],
                          preferred_element_type=jnp.float32)


def _post_body(acc_ref, h_ref, wcomb_ref, e8_ref, g_ref, b_ref, rz1_ref,
               rz2_ref, wff1_ref, wff2_ref, out_ref):
    acc = acc_ref[0] + acc_ref[1]
    uagg = acc[:, :D]
    den = acc[:, D:]
    den = jnp.where(den == 0.0, 1.0, den)
    bcast = jnp.dot(1.0 / den, e8_ref[...], preferred_element_type=jnp.float32)
    agg = uagg * bcast
    mp = jnp.dot(agg, wcomb_ref[...], preferred_element_type=jnp.float32)
    h1 = h_ref[...] + rz1_ref[...] * mp
    mean = jnp.mean(h1, axis=1, keepdims=True)
    var = jnp.mean((h1 - mean) ** 2, axis=1, keepdims=True)
    h1n = g_ref[...] * (h1 - mean) * lax.rsqrt(var + _EPS) + b_ref[...]
    x = jnp.dot(h1n, wff1_ref[...], preferred_element_type=jnp.float32)
    ff = 0.5 * x * (1.0 + lax.erf(x * (2.0 ** -0.5)))
    out_ref[...] = h1 + rz2_ref[...] * jnp.dot(
        ff, wff2_ref[...], preferred_element_type=jnp.float32)


def _sc_attn_body(kv_hbm, q_hbm, ef_hbm, src_hbm, dst_hbm, zero_hbm, out_hbm,
                  srcS, dstS, kv0, kv1, q0, q1, efB, msg0, acc_sh,
                  sg0, sg1, se):
    cid = lax.axis_index("c")
    sid = lax.axis_index("s")
    wid = cid * NS + sid
    r0 = sid * ROWS_PT
    pltpu.sync_copy(zero_hbm.at[pl.ds(r0, ROWS_PT)],
                    acc_sh.at[pl.ds(r0, ROWS_PT)])
    plsc.subcore_barrier()

    pltpu.sync_copy(src_hbm.at[wid].at[pl.ds(0, SB)], srcS.at[0])
    pltpu.sync_copy(dst_hbm.at[wid].at[pl.ds(0, SB)], dstS.at[0])
    pltpu.sync_copy(src_hbm.at[wid].at[pl.ds(SB, SB)], srcS.at[1])
    pltpu.sync_copy(dst_hbm.at[wid].at[pl.ds(SB, SB)], dstS.at[1])

    def src_row(j):
        return srcS.at[(j // SB) % 2].at[j % SB]

    def dst_row(j):
        return dstS.at[(j // SB) % 2].at[j % SB]

    gbufs = ((kv0, q0, sg0), (kv1, q1, sg1))
    ebase = wid * EPW

    def issue_g(j, b):
        kvb, qb, sem = gbufs[b]
        pltpu.async_copy(kv_hbm.at[src_row(j)], kvb, sem)
        pltpu.async_copy(q_hbm.at[dst_row(j)], qb, sem)

    def wait_g(j, b):
        kvb, qb, sem = gbufs[b]
        pltpu.make_async_copy(kv_hbm.at[src_row(j)], kvb, sem).wait()
        pltpu.make_async_copy(q_hbm.at[dst_row(j)], qb, sem).wait()

    def issue_ef(j):
        pltpu.async_copy(ef_hbm.at[pl.ds(ebase + j * C, C)], efB, se)

    def wait_ef(j):
        pltpu.make_async_copy(ef_hbm.at[pl.ds(ebase + j * C, C)], efB,
                              se).wait()

    iota = lax.iota(jnp.int32, 16)
    perms = [iota ^ 8, iota ^ 4, iota ^ 2, iota ^ 1]
    lanesel = [iota == 8 + h for h in range(NH)]
    bidx = [jnp.full((16,), 8 + h, jnp.int32) for h in range(NH)]

    def compute(b, msgB):
        kvb, qb, _ = gbufs[b]

        def edge_body(e, c_):
            sv = jnp.zeros((16,), jnp.float32)
            for h in range(NH):
                kvv = kvb[e, pl.ds(h * KQ, 16)]
                efv = efB[e, pl.ds(h * KQ, 16)]
                qv = qb[e, pl.ds(h * KQ, 16)]
                p = (kvv + efv) * qv
                for pm in perms:
                    p = p + jnp.take(p, pm)
                sv = jnp.where(lanesel[h], p, sv)
            ev = jnp.exp(sv)
            msgB[e, pl.ds(120, 16)] = ev
            for h in range(NH):
                a = jnp.take(ev, bidx[h])
                vv = kvb[e, pl.ds(D + h * MSG, 16)]
                msgB[e, pl.ds(h * MSG, 16)] = a * vv
            return c_

        lax.fori_loop(0, C, edge_body, 0, unroll=4)

    issue_g(0, 0)
    issue_g(1, 1)
    issue_ef(0)

    def pair_body(p, carry):
        for b in range(2):
            j = p * 2 + b

            @pl.when(j < NCHUNK)
            def _():
                wait_g(j, b)
                wait_ef(j)

                compute(b, msg0)
                pltpu.sync_copy(msg0, acc_sh.at[dst_row(j)], add=True)

                nxt = j // SB + 1

                @pl.when(jnp.logical_and(j % SB == 0,
                                         jnp.logical_and(j > 0, nxt < NB)))
                def _():
                    pltpu.sync_copy(src_hbm.at[wid].at[pl.ds(nxt * SB, SB)],
                                    srcS.at[nxt % 2])
                    pltpu.sync_copy(dst_hbm.at[wid].at[pl.ds(nxt * SB, SB)],
                                    dstS.at[nxt % 2])

                @pl.when(j + 1 < NCHUNK)
                def _():
                    issue_ef(j + 1)

                @pl.when(j + 2 < NCHUNK)
                def _():
                    issue_g(j + 2, b)

        return carry

    lax.fori_loop(0, (NCHUNK + 2) // 2, pair_body, 0)
    plsc.subcore_barrier()
    pltpu.sync_copy(acc_sh.at[pl.ds(r0, ROWS_PT)],
                    out_hbm.at[cid].at[pl.ds(r0, ROWS_PT)])


@functools.cache
def _build_sc_attn():
    return pl.kernel(
        _sc_attn_body,
        out_type=jax.ShapeDtypeStruct((NC, VP, ACC_W), jnp.float32),
        mesh=plsc.VectorSubcoreMesh(core_axis_name="c", subcore_axis_name="s",
                                    num_cores=NC, num_subcores=NS),
        compiler_params=pltpu.CompilerParams(use_tc_tiling_on_sc=False,
                                             needs_layout_passes=False),
        scratch_types=[
            pltpu.VMEM((2, SB, C), jnp.int32),
            pltpu.VMEM((2, SB, C), jnp.int32),
            pltpu.VMEM((C, 2 * D), jnp.float32),
            pltpu.VMEM((C, 2 * D), jnp.float32),
            pltpu.VMEM((C, D), jnp.float32),
            pltpu.VMEM((C, D), jnp.float32),
            pltpu.VMEM((C, D), jnp.float32),
            pltpu.VMEM((C, ACC_W), jnp.float32),
            pltpu.VMEM_SHARED((VP, ACC_W), jnp.float32),
            pltpu.SemaphoreType.DMA,
            pltpu.SemaphoreType.DMA,
            pltpu.SemaphoreType.DMA,
        ],
    )


def _sc_attn(kv, q, ef, src, dst, zeros):
    return _build_sc_attn()(kv, q, ef, src, dst, zeros)


def kernel(node_embeddings, adjacency_list, edge_features, W_kqv, bm, ba,
           W_ef, W_comb, g1, b1, g2, b2, rz1, rz2, W_ff1, W_ff2):
    del bm, ba
    f32 = jnp.float32
    w3 = W_kqv.reshape(D, NH, 2 * KQ + MSG)
    wk = w3[:, :, :KQ].reshape(D, NH * KQ)
    wq = w3[:, :, KQ : 2 * KQ].reshape(D, NH * KQ)
    wv = w3[:, :, 2 * KQ :].reshape(D, NH * MSG)
    wkvq = jnp.concatenate([wk, wv, wq], axis=1)

    B1 = 1000
    BE = 2000
    nblk = lambda i: (jnp.minimum(i, V // B1 - 1), 0)
    kv, q, ef = pl.pallas_call(
        _proj_body,
        grid=(E // BE,),
        in_specs=[
            pl.BlockSpec((B1, D), nblk),
            pl.BlockSpec((D, 3 * D), lambda i: (0, 0)),
            pl.BlockSpec((1, D), lambda i: (0, 0)),
            pl.BlockSpec((1, D), lambda i: (0, 0)),
            pl.BlockSpec((BE, DE), lambda i: (i, 0)),
            pl.BlockSpec((DE, NH * KQ), lambda i: (0, 0)),
        ],
        out_specs=[
            pl.BlockSpec((B1, 2 * D), nblk),
            pl.BlockSpec((B1, D), nblk),
            pl.BlockSpec((BE, NH * KQ), lambda i: (i, 0)),
        ],
        out_shape=[
            jax.ShapeDtypeStruct((V, 2 * D), f32),
            jax.ShapeDtypeStruct((V, D), f32),
            jax.ShapeDtypeStruct((E, NH * KQ), f32),
        ],
    )(node_embeddings, wkvq, g1.reshape(1, D), b1.reshape(1, D),
      edge_features, W_ef)

    pad_rows = NB * SB - NCHUNK
    src = jnp.pad(adjacency_list[:, 0].astype(jnp.int32).reshape(
        NW, NCHUNK, C), ((0, 0), (0, pad_rows), (0, 0)))
    dst = jnp.pad(adjacency_list[:, 1].astype(jnp.int32).reshape(
        NW, NCHUNK, C), ((0, 0), (0, pad_rows), (0, 0)))
    zeros = jnp.zeros((VP, ACC_W), f32)
    acc = _sc_attn(kv, q, ef, src, dst, zeros)

    e8 = jnp.kron(jnp.eye(NH, dtype=f32), jnp.ones((1, MSG), f32))
    B3 = 1000
    out = pl.pallas_call(
        _post_body,
        grid=(V // B3,),
        in_specs=[
            pl.BlockSpec((NC, B3, ACC_W), lambda i: (0, i, 0)),
            pl.BlockSpec((B3, D), lambda i: (i, 0)),
            pl.BlockSpec((NH * MSG, D), lambda i: (0, 0)),
            pl.BlockSpec((NH, D), lambda i: (0, 0)),
            pl.BlockSpec((1, D), lambda i: (0, 0)),
            pl.BlockSpec((1, D), lambda i: (0, 0)),
            pl.BlockSpec((1, D), lambda i: (0, 0)),
            pl.BlockSpec((1, D), lambda i: (0, 0)),
            pl.BlockSpec((D, 2 * D), lambda i: (0, 0)),
            pl.BlockSpec((2 * D, D), lambda i: (0, 0)),
        ],
        out_specs=pl.BlockSpec((B3, D), lambda i: (i, 0)),
        out_shape=jax.ShapeDtypeStruct((V, D), f32),
    )(acc, node_embeddings, W_comb, e8, g2.reshape(1, D), b2.reshape(1, D),
      rz1.reshape(1, D), rz2.reshape(1, D), W_ff1, W_ff2)
    return out
